# baseline (speedup 1.0000x reference)
import functools

import jax
import jax.numpy as jnp
from jax import lax
from jax.experimental import pallas as pl
from jax.experimental.pallas import tpu as pltpu
from jax.experimental.pallas import tpu_sc as plsc

_BASE_LEN = 2048
_CH = 24
_NBUF = 4
_DEPTH = 2
_LAG = 2
_NSUB = 16
_BLK = 1032


def _sc_bulk_body(kc, vc, ko, vo, *scratch):
    shared = scratch[:_NBUF]
    in_sems = scratch[_NBUF:2 * _NBUF]
    out_sems = scratch[2 * _NBUF:3 * _NBUF]
    out_len = vo.shape[1]
    half_rows = out_len // 2
    n_chunks = half_rows // _CH

    c = lax.axis_index("c")
    s = lax.axis_index("s")
    b = s
    r0 = c * half_rows
    bufs = tuple(sh.at[s] for sh in shared)

    items = []
    for src_r, dst_r in ((kc, ko), (vc, vo)):
        for i in range(n_chunks):
            items.append((src_r, dst_r, r0 + i * _CH))
    T = len(items)
    in_h = [None] * T
    out_h = [None] * T

    def start_in(t):
        src_r, _, rr = items[t]
        return pltpu.async_copy(
            src_r.at[b, pl.ds(rr, _CH)], bufs[t % _NBUF], in_sems[t % _NBUF]
        )

    def start_out(t):
        _, dst_r, rr = items[t]
        return pltpu.async_copy(
            bufs[t % _NBUF], dst_r.at[b, pl.ds(rr, _CH)], out_sems[t % _NBUF]
        )

    for t in range(min(_DEPTH, T)):
        in_h[t] = start_in(t)
    for t in range(T):
        if t - _LAG >= 0:
            out_h[t - _LAG].wait()
        nxt = t + _DEPTH
        if nxt < T:
            in_h[nxt] = start_in(nxt)
        in_h[t].wait()
        out_h[t] = start_out(t)
    for t in range(max(0, T - _LAG), T):
        out_h[t].wait()


def _tc_bulk_body(kc, ko):
    ko[...] = kc[...]


def _patch_body(pos_ref, kv, vv, _ka, _va, ko, vo, sk, sv):
    seq = kv.shape[1]
    pos = pos_ref[0]
    ck = pltpu.make_async_copy(kv, ko.at[:, pl.ds(pos, seq)], sk)
    cv = pltpu.make_async_copy(vv, vo.at[:, pl.ds(pos, seq)], sv)
    ck.start()
    cv.start()
    ck.wait()
    cv.wait()


def kernel(k_cache, v_cache, input_pos, k_val, v_val):
    bs, seq, n_heads, head_dim = k_val.shape
    out_len = _BASE_LEN + seq
    pos = jnp.asarray(input_pos, dtype=jnp.int32).reshape(1)
    out_sd = jax.ShapeDtypeStruct((bs, out_len, n_heads, head_dim), k_cache.dtype)

    mesh = plsc.VectorSubcoreMesh(core_axis_name="c", subcore_axis_name="s")
    sc_fn = functools.partial(
        pl.kernel,
        out_type=(out_sd, out_sd),
        mesh=mesh,
        scratch_types=(
            [pltpu.VMEM_SHARED((_NSUB, _CH, n_heads, head_dim), k_cache.dtype)
             for _ in range(_NBUF)]
            + [pltpu.SemaphoreType.DMA] * (2 * _NBUF)
        ),
        compiler_params=pltpu.CompilerParams(use_tc_tiling_on_sc=True),
    )(_sc_bulk_body)
    k_bulk, v_bulk = sc_fn(k_cache, v_cache)

    k_out, v_out = pl.pallas_call(
        _patch_body,
        out_shape=(out_sd, out_sd),
        in_specs=[
            pl.BlockSpec(memory_space=pltpu.SMEM),
            pl.BlockSpec(memory_space=pl.ANY),
            pl.BlockSpec(memory_space=pl.ANY),
            pl.BlockSpec(memory_space=pl.ANY),
            pl.BlockSpec(memory_space=pl.ANY),
        ],
        out_specs=(
            pl.BlockSpec(memory_space=pl.ANY),
            pl.BlockSpec(memory_space=pl.ANY),
        ),
        scratch_shapes=[pltpu.SemaphoreType.DMA] * 2,
        input_output_aliases={3: 0, 4: 1},
    )(pos, k_val, v_val, k_bulk, v_bulk)
    return (k_out, v_out)


# speedup vs baseline: 1.0386x; 1.0386x over previous
import functools

import jax
import jax.numpy as jnp
from jax import lax
from jax.experimental import pallas as pl
from jax.experimental.pallas import tpu as pltpu
from jax.experimental.pallas import tpu_sc as plsc

_BASE_LEN = 2048
_CH = 43
_NBUF = 2
_DEPTH = 1
_LAG = 1
_NSUB = 16
_BLK = 1032


def _sc_bulk_body(vc, vo, *scratch):
    shared = scratch[:_NBUF]
    in_sems = scratch[_NBUF:2 * _NBUF]
    out_sems = scratch[2 * _NBUF:3 * _NBUF]
    out_len = vo.shape[1]
    half_rows = out_len // 2
    n_chunks = half_rows // _CH

    c = lax.axis_index("c")
    s = lax.axis_index("s")
    b = s
    r0 = c * half_rows
    bufs = tuple(sh.at[s] for sh in shared)

    in_h = [None] * n_chunks
    out_h = [None] * n_chunks

    def start_in(t):
        return pltpu.async_copy(
            vc.at[b, pl.ds(r0 + t * _CH, _CH)], bufs[t % _NBUF], in_sems[t % _NBUF]
        )

    def start_out(t):
        return pltpu.async_copy(
            bufs[t % _NBUF], vo.at[b, pl.ds(r0 + t * _CH, _CH)], out_sems[t % _NBUF]
        )

    for t in range(min(_DEPTH, n_chunks)):
        in_h[t] = start_in(t)
    for t in range(n_chunks):
        if t - _LAG >= 0:
            out_h[t - _LAG].wait()
        nxt = t + _DEPTH
        if nxt < n_chunks:
            in_h[nxt] = start_in(nxt)
        in_h[t].wait()
        out_h[t] = start_out(t)
    for t in range(max(0, n_chunks - _LAG), n_chunks):
        out_h[t].wait()


def _tc_bulk_body(kc, ko):
    ko[...] = kc[...]


def _patch_body(pos_ref, kv, vv, _ka, _va, ko, vo, sk, sv):
    seq = kv.shape[1]
    pos = pos_ref[0]
    ck = pltpu.make_async_copy(kv, ko.at[:, pl.ds(pos, seq)], sk)
    cv = pltpu.make_async_copy(vv, vo.at[:, pl.ds(pos, seq)], sv)
    ck.start()
    cv.start()
    ck.wait()
    cv.wait()


def kernel(k_cache, v_cache, input_pos, k_val, v_val):
    bs, seq, n_heads, head_dim = k_val.shape
    out_len = _BASE_LEN + seq
    pos = jnp.asarray(input_pos, dtype=jnp.int32).reshape(1)
    out_sd = jax.ShapeDtypeStruct((bs, out_len, n_heads, head_dim), k_cache.dtype)

    n_blk = out_len // _BLK
    blk = (1, _BLK, n_heads, head_dim)
    k_bulk = pl.pallas_call(
        _tc_bulk_body,
        grid=(bs, n_blk),
        out_shape=out_sd,
        in_specs=[pl.BlockSpec(blk, lambda b, i: (b, i, 0, 0))],
        out_specs=pl.BlockSpec(blk, lambda b, i: (b, i, 0, 0)),
    )(k_cache)

    mesh = plsc.VectorSubcoreMesh(core_axis_name="c", subcore_axis_name="s")
    sc_fn = functools.partial(
        pl.kernel,
        out_type=out_sd,
        mesh=mesh,
        scratch_types=(
            [pltpu.VMEM_SHARED((_NSUB, _CH, n_heads, head_dim), k_cache.dtype)
             for _ in range(_NBUF)]
            + [pltpu.SemaphoreType.DMA] * (2 * _NBUF)
        ),
        compiler_params=pltpu.CompilerParams(use_tc_tiling_on_sc=True),
    )(_sc_bulk_body)
    v_bulk = sc_fn(v_cache)

    k_out, v_out = pl.pallas_call(
        _patch_body,
        out_shape=(out_sd, out_sd),
        in_specs=[
            pl.BlockSpec(memory_space=pltpu.SMEM),
            pl.BlockSpec(memory_space=pl.ANY),
            pl.BlockSpec(memory_space=pl.ANY),
            pl.BlockSpec(memory_space=pl.ANY),
            pl.BlockSpec(memory_space=pl.ANY),
        ],
        out_specs=(
            pl.BlockSpec(memory_space=pl.ANY),
            pl.BlockSpec(memory_space=pl.ANY),
        ),
        scratch_shapes=[pltpu.SemaphoreType.DMA] * 2,
        input_output_aliases={3: 0, 4: 1},
    )(pos, k_val, v_val, k_bulk, v_bulk)
    return (k_out, v_out)
